# Initial kernel scaffold; baseline (speedup 1.0000x reference)
import jax
import jax.numpy as jnp
from jax.experimental import pallas as pl


def kernel(batchinput_tensor, grapharea_matrix, X, W_ih_0, W_hh_0, b_ih_0, b_hh_0, W_ih_1, W_hh_1, b_ih_1, b_hh_1, W_ih_2, W_hh_2, b_ih_2, b_hh_2, W_ih_s, W_hh_s, b_ih_s, b_hh_s, Wg, bg, Ws, bs, memory_hn, memory_hn_senses):
    raise NotImplementedError("write your pallas kernel here")


# speedup vs baseline: 1.2952x; 1.2952x over previous
import functools

import jax
import jax.numpy as jnp
from jax import lax
from jax.experimental import pallas as pl
from jax.experimental.pallas import tpu as pltpu
from jax.experimental.pallas import tpu_sc as plsc

NUM_NODES = 60000
D = 300
B = 32
S = 35
H = 1150
R = B * S
NEG = -1e30


def _sc_gather(table, idx_padded, n_rows, n_cols):
    info = plsc.get_sparse_core_info()
    nw = info.num_cores * info.num_subcores
    per_w = n_rows // nw
    mesh = plsc.VectorSubcoreMesh(core_axis_name="c", subcore_axis_name="s")

    @functools.partial(
        pl.kernel,
        mesh=mesh,
        out_type=jax.ShapeDtypeStruct((n_rows, n_cols), jnp.float32),
        scratch_types=[
            pltpu.VMEM((per_w,), jnp.int32),
            pltpu.VMEM((per_w, n_cols), jnp.float32),
            pltpu.SemaphoreType.DMA,
        ],
    )
    def k(table_hbm, idx_hbm, out_hbm, idx_v, rows_v, sem):
        wid = lax.axis_index("s") * info.num_cores + lax.axis_index("c")
        base = wid * per_w
        pltpu.sync_copy(idx_hbm.at[pl.ds(base, per_w)], idx_v)
        pltpu.async_copy(table_hbm.at[idx_v], rows_v, sem).wait()
        pltpu.sync_copy(rows_v, out_hbm.at[pl.ds(base, per_w)])

    return k(table, idx_padded)


def _dot(a, b):
    return lax.dot_general(a, b, (((1,), (1,)), ((), ())),
                           preferred_element_type=jnp.float32)


def _gru_body(x_ref, h0_ref, wir, wiz, win, whr, whz, whn, bi_ref, bh_ref,
              out_ref, gir, giz, gin, h_s):
    x = x_ref[...]
    gir[...] = _dot(x, wir[...]) + bi_ref[0:1, :]
    giz[...] = _dot(x, wiz[...]) + bi_ref[1:2, :]
    gin[...] = _dot(x, win[...]) + bi_ref[2:3, :]
    h_s[...] = h0_ref[...]

    def step(t, carry):
        h = h_s[...]
        hb = h.astype(jnp.bfloat16)
        ghr = _dot(hb, whr[...]) + bh_ref[0:1, :]
        ghz = _dot(hb, whz[...]) + bh_ref[1:2, :]
        ghn = _dot(hb, whn[...]) + bh_ref[2:3, :]
        row = pl.multiple_of(t * B, B)
        r = jax.nn.sigmoid(gir[pl.ds(row, B), :] + ghr)
        z = jax.nn.sigmoid(giz[pl.ds(row, B), :] + ghz)
        n = jnp.tanh(gin[pl.ds(row, B), :] + r * ghn)
        hn = (1.0 - z) * n + z * h
        h_s[...] = hn
        out_ref[pl.ds(row, B), :] = hn.astype(jnp.bfloat16)
        return carry

    lax.fori_loop(0, S, step, 0)


def _gru_layer(x_bf, h0, w_ih, w_hh, b_ih, b_hh):
    n_in = x_bf.shape[1]
    wi = w_ih.astype(jnp.bfloat16)
    wh = w_hh.astype(jnp.bfloat16)
    wir, wiz, win = wi[0:H], wi[H:2 * H], wi[2 * H:3 * H]
    whr, whz, whn = wh[0:H], wh[H:2 * H], wh[2 * H:3 * H]
    bi = b_ih.reshape(3, H)
    bh = b_hh.reshape(3, H)
    return pl.pallas_call(
        _gru_body,
        out_shape=jax.ShapeDtypeStruct((R, H), jnp.bfloat16),
        scratch_shapes=[
            pltpu.VMEM((R, H), jnp.float32),
            pltpu.VMEM((R, H), jnp.float32),
            pltpu.VMEM((R, H), jnp.float32),
            pltpu.VMEM((B, H), jnp.float32),
        ],
    )(x_bf, h0, wir, wiz, win, whr, whz, whn, bi, bh)


VT = 1024


def _proj_body(v_total, n_tiles, h_ref, w_ref, b_ref, logit_ref, lse_ref,
               m_s, s_s):
    i = pl.program_id(0)

    @pl.when(i == 0)
    def _():
        m_s[...] = jnp.full((R, 1), NEG, jnp.float32)
        s_s[...] = jnp.zeros((R, 1), jnp.float32)

    w = w_ref[...].astype(jnp.bfloat16)
    logits = _dot(h_ref[...], w) + b_ref[...]
    col = lax.broadcasted_iota(jnp.int32, (1, VT), 1) + i * VT
    masked = jnp.where(col < v_total, logits, NEG)
    tmax = jnp.max(masked, axis=1, keepdims=True)
    m_old = m_s[...]
    s_old = s_s[...]
    m_new = jnp.maximum(m_old, tmax)
    s_new = s_old * jnp.exp(m_old - m_new) + jnp.sum(
        jnp.exp(masked - m_new), axis=1, keepdims=True)
    m_s[...] = m_new
    s_s[...] = s_new
    logit_ref[...] = logits.astype(jnp.bfloat16)

    @pl.when(i == n_tiles - 1)
    def _():
        lse_ref[...] = m_new + jnp.log(s_new)


def _norm_body(logit_ref, lse_ref, out_ref):
    out_ref[...] = logit_ref[...].astype(jnp.float32) - lse_ref[...]


def _proj_log_softmax(h_bf, w, b):
    v_total = w.shape[0]
    n_tiles = pl.cdiv(v_total, VT)
    b2 = b.reshape(1, v_total)
    logits, lse = pl.pallas_call(
        functools.partial(_proj_body, v_total, n_tiles),
        grid=(n_tiles,),
        in_specs=[
            pl.BlockSpec((R, H), lambda i: (0, 0)),
            pl.BlockSpec((VT, H), lambda i: (i, 0)),
            pl.BlockSpec((1, VT), lambda i: (0, i)),
        ],
        out_specs=[
            pl.BlockSpec((R, VT), lambda i: (0, i)),
            pl.BlockSpec((R, 1), lambda i: (0, 0)),
        ],
        out_shape=[
            jax.ShapeDtypeStruct((R, v_total), jnp.bfloat16),
            jax.ShapeDtypeStruct((R, 1), jnp.float32),
        ],
        scratch_shapes=[
            pltpu.VMEM((R, 1), jnp.float32),
            pltpu.VMEM((R, 1), jnp.float32),
        ],
    )(h_bf, w, b2)
    return pl.pallas_call(
        _norm_body,
        grid=(n_tiles,),
        in_specs=[
            pl.BlockSpec((R, VT), lambda i: (0, i)),
            pl.BlockSpec((R, 1), lambda i: (0, 0)),
        ],
        out_specs=pl.BlockSpec((R, VT), lambda i: (0, i)),
        out_shape=jax.ShapeDtypeStruct((R, v_total), jnp.float32),
    )(logits, lse)


def kernel(batchinput_tensor, grapharea_matrix, X,
           W_ih_0, W_hh_0, b_ih_0, b_hh_0,
           W_ih_1, W_hh_1, b_ih_1, b_hh_1,
           W_ih_2, W_hh_2, b_ih_2, b_hh_2,
           W_ih_s, W_hh_s, b_ih_s, b_hh_s,
           Wg, bg, Ws, bs, memory_hn, memory_hn_senses):
    word_idx = batchinput_tensor[:, :, 0, 0].astype(jnp.int32)
    idx_sb = word_idx.T.reshape(-1)
    idx_pad = jnp.concatenate([idx_sb, jnp.zeros((1280 - R,), jnp.int32)])
    X_pad = jnp.pad(X, ((0, 0), (0, 384 - D)))
    emb = _sc_gather(X_pad, idx_pad, 1280, 384)[:R, :D]

    x = emb.astype(jnp.bfloat16)
    out0 = _gru_layer(x, memory_hn[0], W_ih_0, W_hh_0, b_ih_0, b_hh_0)
    out1 = _gru_layer(out0, memory_hn[1], W_ih_1, W_hh_1, b_ih_1, b_hh_1)
    out2 = _gru_layer(out1, memory_hn[2], W_ih_2, W_hh_2, b_ih_2, b_hh_2)
    outs = _gru_layer(out0, memory_hn_senses[0], W_ih_s, W_hh_s, b_ih_s,
                      b_hh_s)

    main_flat = out2.reshape(S, B, H).transpose(1, 0, 2).reshape(R, H)
    predictions_globals = _proj_log_softmax(main_flat, Wg, bg)
    predictions_senses = _proj_log_softmax(outs, Ws, bs)
    return (predictions_globals, predictions_senses)


# speedup vs baseline: 1.5453x; 1.1931x over previous
import functools

import jax
import jax.numpy as jnp
from jax import lax
from jax.experimental import pallas as pl
from jax.experimental.pallas import tpu as pltpu
from jax.experimental.pallas import tpu_sc as plsc

NUM_NODES = 60000
D = 300
B = 32
S = 35
H = 1150
R = B * S
NEG = -1e30


def _sc_gather(table, idx_padded, n_rows, n_cols):
    info = plsc.get_sparse_core_info()
    nw = info.num_cores * info.num_subcores
    per_w = n_rows // nw
    mesh = plsc.VectorSubcoreMesh(core_axis_name="c", subcore_axis_name="s")

    @functools.partial(
        pl.kernel,
        mesh=mesh,
        out_type=jax.ShapeDtypeStruct((n_rows, n_cols), jnp.float32),
        scratch_types=[
            pltpu.VMEM((per_w,), jnp.int32),
            pltpu.VMEM((per_w, n_cols), jnp.float32),
            pltpu.SemaphoreType.DMA,
        ],
    )
    def k(table_hbm, idx_hbm, out_hbm, idx_v, rows_v, sem):
        wid = lax.axis_index("s") * info.num_cores + lax.axis_index("c")
        base = wid * per_w
        pltpu.sync_copy(idx_hbm.at[pl.ds(base, per_w)], idx_v)
        pltpu.async_copy(table_hbm.at[idx_v], rows_v, sem).wait()
        pltpu.sync_copy(rows_v, out_hbm.at[pl.ds(base, per_w)])

    return k(table, idx_padded)


PAD_RT = 2000


def _pad_body(x_ref, out_ref):
    x = x_ref[...]
    out_ref[...] = jnp.concatenate(
        [x, jnp.zeros((PAD_RT, 384 - D), jnp.float32)], axis=1)


def _pad_table(X):
    n = X.shape[0]
    return pl.pallas_call(
        _pad_body,
        grid=(n // PAD_RT,),
        in_specs=[pl.BlockSpec((PAD_RT, D), lambda i: (i, 0))],
        out_specs=pl.BlockSpec((PAD_RT, 384), lambda i: (i, 0)),
        out_shape=jax.ShapeDtypeStruct((n, 384), jnp.float32),
    )(X)


def _dot(a, b):
    return lax.dot_general(a, b, (((1,), (1,)), ((), ())),
                           preferred_element_type=jnp.float32)


def _gru_body(x_ref, h0_ref, wir, wiz, win, whr, whz, whn, bi_ref, bh_ref,
              out_ref, gir, giz, gin, h_s):
    x = x_ref[...]
    gir[...] = _dot(x, wir[...]) + bi_ref[0:1, :]
    giz[...] = _dot(x, wiz[...]) + bi_ref[1:2, :]
    gin[...] = _dot(x, win[...]) + bi_ref[2:3, :]
    h_s[...] = h0_ref[...]

    def step(t, carry):
        h = h_s[...]
        hb = h.astype(jnp.bfloat16)
        ghr = _dot(hb, whr[...]) + bh_ref[0:1, :]
        ghz = _dot(hb, whz[...]) + bh_ref[1:2, :]
        ghn = _dot(hb, whn[...]) + bh_ref[2:3, :]
        row = pl.multiple_of(t * B, B)
        r = jax.nn.sigmoid(gir[pl.ds(row, B), :] + ghr)
        z = jax.nn.sigmoid(giz[pl.ds(row, B), :] + ghz)
        n = jnp.tanh(gin[pl.ds(row, B), :] + r * ghn)
        hn = (1.0 - z) * n + z * h
        h_s[...] = hn
        out_ref[pl.ds(row, B), :] = hn.astype(jnp.bfloat16)
        return carry

    lax.fori_loop(0, S, step, 0)


def _gru_layer(x_bf, h0, w_ih, w_hh, b_ih, b_hh):
    n_in = x_bf.shape[1]
    wi = w_ih.astype(jnp.bfloat16)
    wh = w_hh.astype(jnp.bfloat16)
    wir, wiz, win = wi[0:H], wi[H:2 * H], wi[2 * H:3 * H]
    whr, whz, whn = wh[0:H], wh[H:2 * H], wh[2 * H:3 * H]
    bi = b_ih.reshape(3, H)
    bh = b_hh.reshape(3, H)
    return pl.pallas_call(
        _gru_body,
        out_shape=jax.ShapeDtypeStruct((R, H), jnp.bfloat16),
        scratch_shapes=[
            pltpu.VMEM((R, H), jnp.float32),
            pltpu.VMEM((R, H), jnp.float32),
            pltpu.VMEM((R, H), jnp.float32),
            pltpu.VMEM((B, H), jnp.float32),
        ],
    )(x_bf, h0, wir, wiz, win, whr, whz, whn, bi, bh)


VT = 1024


def _proj_body(v_total, n_tiles, h_ref, w_ref, b_ref, logit_ref, lse_ref,
               m_s, s_s):
    i = pl.program_id(0)

    @pl.when(i == 0)
    def _():
        m_s[...] = jnp.full((R, 1), NEG, jnp.float32)
        s_s[...] = jnp.zeros((R, 1), jnp.float32)

    w = w_ref[...].astype(jnp.bfloat16)
    logits = _dot(h_ref[...], w) + b_ref[...]
    col = lax.broadcasted_iota(jnp.int32, (1, VT), 1) + i * VT
    masked = jnp.where(col < v_total, logits, NEG)
    tmax = jnp.max(masked, axis=1, keepdims=True)
    m_old = m_s[...]
    s_old = s_s[...]
    m_new = jnp.maximum(m_old, tmax)
    s_new = s_old * jnp.exp(m_old - m_new) + jnp.sum(
        jnp.exp(masked - m_new), axis=1, keepdims=True)
    m_s[...] = m_new
    s_s[...] = s_new
    logit_ref[...] = logits.astype(jnp.bfloat16)

    @pl.when(i == n_tiles - 1)
    def _():
        lse_ref[...] = m_new + jnp.log(s_new)


def _norm_body(logit_ref, lse_ref, out_ref):
    out_ref[...] = logit_ref[...].astype(jnp.float32) - lse_ref[...]


def _proj_log_softmax(h_bf, w, b):
    v_total = w.shape[0]
    n_tiles = pl.cdiv(v_total, VT)
    b2 = b.reshape(1, v_total)
    logits, lse = pl.pallas_call(
        functools.partial(_proj_body, v_total, n_tiles),
        grid=(n_tiles,),
        in_specs=[
            pl.BlockSpec((R, H), lambda i: (0, 0)),
            pl.BlockSpec((VT, H), lambda i: (i, 0)),
            pl.BlockSpec((1, VT), lambda i: (0, i)),
        ],
        out_specs=[
            pl.BlockSpec((R, VT), lambda i: (0, i)),
            pl.BlockSpec((R, 1), lambda i: (0, 0)),
        ],
        out_shape=[
            jax.ShapeDtypeStruct((R, v_total), jnp.bfloat16),
            jax.ShapeDtypeStruct((R, 1), jnp.float32),
        ],
        scratch_shapes=[
            pltpu.VMEM((R, 1), jnp.float32),
            pltpu.VMEM((R, 1), jnp.float32),
        ],
    )(h_bf, w, b2)
    return pl.pallas_call(
        _norm_body,
        grid=(n_tiles,),
        in_specs=[
            pl.BlockSpec((R, VT), lambda i: (0, i)),
            pl.BlockSpec((R, 1), lambda i: (0, 0)),
        ],
        out_specs=pl.BlockSpec((R, VT), lambda i: (0, i)),
        out_shape=jax.ShapeDtypeStruct((R, v_total), jnp.float32),
    )(logits, lse)


def kernel(batchinput_tensor, grapharea_matrix, X,
           W_ih_0, W_hh_0, b_ih_0, b_hh_0,
           W_ih_1, W_hh_1, b_ih_1, b_hh_1,
           W_ih_2, W_hh_2, b_ih_2, b_hh_2,
           W_ih_s, W_hh_s, b_ih_s, b_hh_s,
           Wg, bg, Ws, bs, memory_hn, memory_hn_senses):
    word_idx = batchinput_tensor[:, :, 0, 0].astype(jnp.int32)
    idx_sb = word_idx.T.reshape(-1)
    idx_pad = jnp.concatenate([idx_sb, jnp.zeros((1280 - R,), jnp.int32)])
    X_pad = _pad_table(X)
    emb = _sc_gather(X_pad, idx_pad, 1280, 384)[:R, :D]

    x = emb.astype(jnp.bfloat16)
    out0 = _gru_layer(x, memory_hn[0], W_ih_0, W_hh_0, b_ih_0, b_hh_0)
    out1 = _gru_layer(out0, memory_hn[1], W_ih_1, W_hh_1, b_ih_1, b_hh_1)
    out2 = _gru_layer(out1, memory_hn[2], W_ih_2, W_hh_2, b_ih_2, b_hh_2)
    outs = _gru_layer(out0, memory_hn_senses[0], W_ih_s, W_hh_s, b_ih_s,
                      b_hh_s)

    main_flat = out2.reshape(S, B, H).transpose(1, 0, 2).reshape(R, H)
    predictions_globals = _proj_log_softmax(main_flat, Wg, bg)
    predictions_senses = _proj_log_softmax(outs, Ws, bs)
    return (predictions_globals, predictions_senses)


# speedup vs baseline: 1.5640x; 1.0121x over previous
import functools

import jax
import jax.numpy as jnp
from jax import lax
from jax.experimental import pallas as pl
from jax.experimental.pallas import tpu as pltpu
from jax.experimental.pallas import tpu_sc as plsc

NUM_NODES = 60000
D = 300
B = 32
S = 35
H = 1150
R = B * S
NEG = -1e30


def _sc_gather(table, idx_padded, n_rows, n_cols):
    info = plsc.get_sparse_core_info()
    nw = info.num_cores * info.num_subcores
    per_w = n_rows // nw
    mesh = plsc.VectorSubcoreMesh(core_axis_name="c", subcore_axis_name="s")

    @functools.partial(
        pl.kernel,
        mesh=mesh,
        out_type=jax.ShapeDtypeStruct((n_rows, n_cols), jnp.float32),
        scratch_types=[
            pltpu.VMEM((per_w,), jnp.int32),
            pltpu.VMEM((per_w, n_cols), jnp.float32),
            pltpu.SemaphoreType.DMA,
        ],
    )
    def k(table_hbm, idx_hbm, out_hbm, idx_v, rows_v, sem):
        wid = lax.axis_index("s") * info.num_cores + lax.axis_index("c")
        base = wid * per_w
        pltpu.sync_copy(idx_hbm.at[pl.ds(base, per_w)], idx_v)
        pltpu.async_copy(table_hbm.at[idx_v], rows_v, sem).wait()
        pltpu.sync_copy(rows_v, out_hbm.at[pl.ds(base, per_w)])

    return k(table, idx_padded)


PAD_RT = 2000


def _pad_body(x_ref, out_ref):
    x = x_ref[...]
    out_ref[...] = jnp.concatenate(
        [x, jnp.zeros((PAD_RT, 384 - D), jnp.float32)], axis=1)


def _pad_table(X):
    n = X.shape[0]
    return pl.pallas_call(
        _pad_body,
        grid=(n // PAD_RT,),
        in_specs=[pl.BlockSpec((PAD_RT, D), lambda i: (i, 0))],
        out_specs=pl.BlockSpec((PAD_RT, 384), lambda i: (i, 0)),
        out_shape=jax.ShapeDtypeStruct((n, 384), jnp.float32),
    )(X)


def _dot(a, b):
    return lax.dot_general(a, b, (((1,), (1,)), ((), ())),
                           preferred_element_type=jnp.float32)


def _gru_body(x_ref, h0_ref, wi_ref, wh_ref, bi_ref, bh_ref,
              out_ref, g3_s, h_s):
    g3_s[...] = _dot(x_ref[...], wi_ref[...]) + bi_ref[...]
    h_s[...] = h0_ref[...]

    def step(t, carry):
        h = h_s[...]
        hb = h.astype(jnp.bfloat16)
        gh = _dot(hb, wh_ref[...]) + bh_ref[...]
        row = pl.multiple_of(t * B, B)
        gi = g3_s[pl.ds(row, B), :]
        r = jax.nn.sigmoid(gi[:, 0:H] + gh[:, 0:H])
        z = jax.nn.sigmoid(gi[:, H:2 * H] + gh[:, H:2 * H])
        n = jnp.tanh(gi[:, 2 * H:3 * H] + r * gh[:, 2 * H:3 * H])
        hn = (1.0 - z) * n + z * h
        h_s[...] = hn
        out_ref[pl.ds(row, B), :] = hn.astype(jnp.bfloat16)
        return carry

    lax.fori_loop(0, S, step, 0)


def _gru_layer(x_bf, h0, w_ih, w_hh, b_ih, b_hh):
    wi = w_ih.astype(jnp.bfloat16)
    wh = w_hh.astype(jnp.bfloat16)
    bi = b_ih.reshape(1, 3 * H)
    bh = b_hh.reshape(1, 3 * H)
    return pl.pallas_call(
        _gru_body,
        out_shape=jax.ShapeDtypeStruct((R, H), jnp.bfloat16),
        scratch_shapes=[
            pltpu.VMEM((R, 3 * H), jnp.float32),
            pltpu.VMEM((B, H), jnp.float32),
        ],
    )(x_bf, h0, wi, wh, bi, bh)


VT = 1024


def _proj_body(v_total, n_tiles, h_ref, w_ref, b_ref, logit_ref, lse_ref,
               m_s, s_s):
    i = pl.program_id(0)

    @pl.when(i == 0)
    def _():
        m_s[...] = jnp.full((R, 1), NEG, jnp.float32)
        s_s[...] = jnp.zeros((R, 1), jnp.float32)

    w = w_ref[...].astype(jnp.bfloat16)
    logits = _dot(h_ref[...], w) + b_ref[...]
    col = lax.broadcasted_iota(jnp.int32, (1, VT), 1) + i * VT
    masked = jnp.where(col < v_total, logits, NEG)
    tmax = jnp.max(masked, axis=1, keepdims=True)
    m_old = m_s[...]
    s_old = s_s[...]
    m_new = jnp.maximum(m_old, tmax)
    s_new = s_old * jnp.exp(m_old - m_new) + jnp.sum(
        jnp.exp(masked - m_new), axis=1, keepdims=True)
    m_s[...] = m_new
    s_s[...] = s_new
    logit_ref[...] = logits.astype(jnp.bfloat16)

    @pl.when(i == n_tiles - 1)
    def _():
        lse_ref[...] = m_new + jnp.log(s_new)


def _norm_body(logit_ref, lse_ref, out_ref):
    out_ref[...] = logit_ref[...].astype(jnp.float32) - lse_ref[...]


def _proj_log_softmax(h_bf, w, b):
    v_total = w.shape[0]
    n_tiles = pl.cdiv(v_total, VT)
    b2 = b.reshape(1, v_total)
    logits, lse = pl.pallas_call(
        functools.partial(_proj_body, v_total, n_tiles),
        grid=(n_tiles,),
        in_specs=[
            pl.BlockSpec((R, H), lambda i: (0, 0)),
            pl.BlockSpec((VT, H), lambda i: (i, 0)),
            pl.BlockSpec((1, VT), lambda i: (0, i)),
        ],
        out_specs=[
            pl.BlockSpec((R, VT), lambda i: (0, i)),
            pl.BlockSpec((R, 1), lambda i: (0, 0)),
        ],
        out_shape=[
            jax.ShapeDtypeStruct((R, v_total), jnp.bfloat16),
            jax.ShapeDtypeStruct((R, 1), jnp.float32),
        ],
        scratch_shapes=[
            pltpu.VMEM((R, 1), jnp.float32),
            pltpu.VMEM((R, 1), jnp.float32),
        ],
    )(h_bf, w, b2)
    return pl.pallas_call(
        _norm_body,
        grid=(n_tiles,),
        in_specs=[
            pl.BlockSpec((R, VT), lambda i: (0, i)),
            pl.BlockSpec((R, 1), lambda i: (0, 0)),
        ],
        out_specs=pl.BlockSpec((R, VT), lambda i: (0, i)),
        out_shape=jax.ShapeDtypeStruct((R, v_total), jnp.float32),
    )(logits, lse)


def kernel(batchinput_tensor, grapharea_matrix, X,
           W_ih_0, W_hh_0, b_ih_0, b_hh_0,
           W_ih_1, W_hh_1, b_ih_1, b_hh_1,
           W_ih_2, W_hh_2, b_ih_2, b_hh_2,
           W_ih_s, W_hh_s, b_ih_s, b_hh_s,
           Wg, bg, Ws, bs, memory_hn, memory_hn_senses):
    word_idx = batchinput_tensor[:, :, 0, 0].astype(jnp.int32)
    idx_sb = word_idx.T.reshape(-1)
    idx_pad = jnp.concatenate([idx_sb, jnp.zeros((1280 - R,), jnp.int32)])
    X_pad = _pad_table(X)
    emb = _sc_gather(X_pad, idx_pad, 1280, 384)[:R, :D]

    x = emb.astype(jnp.bfloat16)
    out0 = _gru_layer(x, memory_hn[0], W_ih_0, W_hh_0, b_ih_0, b_hh_0)
    out1 = _gru_layer(out0, memory_hn[1], W_ih_1, W_hh_1, b_ih_1, b_hh_1)
    out2 = _gru_layer(out1, memory_hn[2], W_ih_2, W_hh_2, b_ih_2, b_hh_2)
    outs = _gru_layer(out0, memory_hn_senses[0], W_ih_s, W_hh_s, b_ih_s,
                      b_hh_s)

    main_flat = out2.reshape(S, B, H).transpose(1, 0, 2).reshape(R, H)
    predictions_globals = _proj_log_softmax(main_flat, Wg, bg)
    predictions_senses = _proj_log_softmax(outs, Ws, bs)
    return (predictions_globals, predictions_senses)


# speedup vs baseline: 1.7159x; 1.0971x over previous
import functools

import jax
import jax.numpy as jnp
from jax import lax
from jax.experimental import pallas as pl
from jax.experimental.pallas import tpu as pltpu
from jax.experimental.pallas import tpu_sc as plsc

NUM_NODES = 60000
D = 300
B = 32
S = 35
H = 1150
R = B * S
NEG = -1e30


def _sc_gather(table, idx_padded, n_rows, n_cols):
    info = plsc.get_sparse_core_info()
    nw = info.num_cores * info.num_subcores
    per_w = n_rows // nw
    mesh = plsc.VectorSubcoreMesh(core_axis_name="c", subcore_axis_name="s")

    @functools.partial(
        pl.kernel,
        mesh=mesh,
        out_type=jax.ShapeDtypeStruct((n_rows, n_cols), jnp.float32),
        scratch_types=[
            pltpu.VMEM((per_w,), jnp.int32),
            pltpu.VMEM((per_w, n_cols), jnp.float32),
            pltpu.SemaphoreType.DMA,
        ],
    )
    def k(table_hbm, idx_hbm, out_hbm, idx_v, rows_v, sem):
        wid = lax.axis_index("s") * info.num_cores + lax.axis_index("c")
        base = wid * per_w
        pltpu.sync_copy(idx_hbm.at[pl.ds(base, per_w)], idx_v)
        pltpu.async_copy(table_hbm.at[idx_v], rows_v, sem).wait()
        pltpu.sync_copy(rows_v, out_hbm.at[pl.ds(base, per_w)])

    return k(table, idx_padded)


PAD_RT = 2000


def _pad_body(x_ref, out_ref):
    x = x_ref[...]
    out_ref[...] = jnp.concatenate(
        [x, jnp.zeros((PAD_RT, 384 - D), jnp.float32)], axis=1)


def _pad_table(X):
    n = X.shape[0]
    return pl.pallas_call(
        _pad_body,
        grid=(n // PAD_RT,),
        in_specs=[pl.BlockSpec((PAD_RT, D), lambda i: (i, 0))],
        out_specs=pl.BlockSpec((PAD_RT, 384), lambda i: (i, 0)),
        out_shape=jax.ShapeDtypeStruct((n, 384), jnp.float32),
    )(X)


def _dot(a, b):
    return lax.dot_general(a, b, (((1,), (1,)), ((), ())),
                           preferred_element_type=jnp.float32)


def _dotn(a, b):
    return lax.dot_general(a, b, (((1,), (0,)), ((), ())),
                           preferred_element_type=jnp.float32)


def _gru_body(x_ref, h0_ref, wi_ref, wh_ref, bi_ref, bh_ref,
              out_ref, g3_s, h_s):
    g3_s[...] = _dotn(x_ref[...], wi_ref[...]) + bi_ref[...]
    h_s[...] = h0_ref[...]

    def step(t, carry):
        h = h_s[...]
        hb = h.astype(jnp.bfloat16)
        gh = _dotn(hb, wh_ref[...]) + bh_ref[...]
        row = pl.multiple_of(t * B, B)
        gi = g3_s[pl.ds(row, B), :]
        r = jax.nn.sigmoid(gi[:, 0:H] + gh[:, 0:H])
        z = jax.nn.sigmoid(gi[:, H:2 * H] + gh[:, H:2 * H])
        n = jnp.tanh(gi[:, 2 * H:3 * H] + r * gh[:, 2 * H:3 * H])
        hn = (1.0 - z) * n + z * h
        h_s[...] = hn
        out_ref[pl.ds(row, B), :] = hn.astype(jnp.bfloat16)
        return carry

    lax.fori_loop(0, S, step, 0)


def _gru_layer(x_bf, h0, w_ih, w_hh, b_ih, b_hh):
    wi = w_ih.T.astype(jnp.bfloat16)
    wh = w_hh.T.astype(jnp.bfloat16)
    bi = b_ih.reshape(1, 3 * H)
    bh = b_hh.reshape(1, 3 * H)
    return pl.pallas_call(
        _gru_body,
        out_shape=jax.ShapeDtypeStruct((R, H), jnp.bfloat16),
        scratch_shapes=[
            pltpu.VMEM((R, 3 * H), jnp.float32),
            pltpu.VMEM((B, H), jnp.float32),
        ],
    )(x_bf, h0, wi, wh, bi, bh)


VT = 1024


def _proj_body(v_total, n_tiles, h_ref, w_ref, b_ref, logit_ref, lse_ref,
               m_s, s_s):
    i = pl.program_id(0)

    @pl.when(i == 0)
    def _():
        m_s[...] = jnp.full((R, 1), NEG, jnp.float32)
        s_s[...] = jnp.zeros((R, 1), jnp.float32)

    w = w_ref[...].astype(jnp.bfloat16)
    logits = _dot(h_ref[...], w) + b_ref[...]
    col = lax.broadcasted_iota(jnp.int32, (1, VT), 1) + i * VT
    masked = jnp.where(col < v_total, logits, NEG)
    tmax = jnp.max(masked, axis=1, keepdims=True)
    m_old = m_s[...]
    s_old = s_s[...]
    m_new = jnp.maximum(m_old, tmax)
    s_new = s_old * jnp.exp(m_old - m_new) + jnp.sum(
        jnp.exp(masked - m_new), axis=1, keepdims=True)
    m_s[...] = m_new
    s_s[...] = s_new
    logit_ref[...] = logits.astype(jnp.bfloat16)

    @pl.when(i == n_tiles - 1)
    def _():
        lse_ref[...] = m_new + jnp.log(s_new)


def _norm_body(logit_ref, lse_ref, out_ref):
    out_ref[...] = logit_ref[...].astype(jnp.float32) - lse_ref[...]


def _proj_log_softmax(h_bf, w, b):
    v_total = w.shape[0]
    n_tiles = pl.cdiv(v_total, VT)
    b2 = b.reshape(1, v_total)
    logits, lse = pl.pallas_call(
        functools.partial(_proj_body, v_total, n_tiles),
        grid=(n_tiles,),
        in_specs=[
            pl.BlockSpec((R, H), lambda i: (0, 0)),
            pl.BlockSpec((VT, H), lambda i: (i, 0)),
            pl.BlockSpec((1, VT), lambda i: (0, i)),
        ],
        out_specs=[
            pl.BlockSpec((R, VT), lambda i: (0, i)),
            pl.BlockSpec((R, 1), lambda i: (0, 0)),
        ],
        out_shape=[
            jax.ShapeDtypeStruct((R, v_total), jnp.bfloat16),
            jax.ShapeDtypeStruct((R, 1), jnp.float32),
        ],
        scratch_shapes=[
            pltpu.VMEM((R, 1), jnp.float32),
            pltpu.VMEM((R, 1), jnp.float32),
        ],
    )(h_bf, w, b2)
    return pl.pallas_call(
        _norm_body,
        grid=(n_tiles,),
        in_specs=[
            pl.BlockSpec((R, VT), lambda i: (0, i)),
            pl.BlockSpec((R, 1), lambda i: (0, 0)),
        ],
        out_specs=pl.BlockSpec((R, VT), lambda i: (0, i)),
        out_shape=jax.ShapeDtypeStruct((R, v_total), jnp.float32),
    )(logits, lse)


def kernel(batchinput_tensor, grapharea_matrix, X,
           W_ih_0, W_hh_0, b_ih_0, b_hh_0,
           W_ih_1, W_hh_1, b_ih_1, b_hh_1,
           W_ih_2, W_hh_2, b_ih_2, b_hh_2,
           W_ih_s, W_hh_s, b_ih_s, b_hh_s,
           Wg, bg, Ws, bs, memory_hn, memory_hn_senses):
    word_idx = batchinput_tensor[:, :, 0, 0].astype(jnp.int32)
    idx_sb = word_idx.T.reshape(-1)
    idx_pad = jnp.concatenate([idx_sb, jnp.zeros((1280 - R,), jnp.int32)])
    X_pad = _pad_table(X)
    emb = _sc_gather(X_pad, idx_pad, 1280, 384)[:R, :D]

    x = emb.astype(jnp.bfloat16)
    out0 = _gru_layer(x, memory_hn[0], W_ih_0, W_hh_0, b_ih_0, b_hh_0)
    out1 = _gru_layer(out0, memory_hn[1], W_ih_1, W_hh_1, b_ih_1, b_hh_1)
    out2 = _gru_layer(out1, memory_hn[2], W_ih_2, W_hh_2, b_ih_2, b_hh_2)
    outs = _gru_layer(out0, memory_hn_senses[0], W_ih_s, W_hh_s, b_ih_s,
                      b_hh_s)

    main_flat = out2.reshape(S, B, H).transpose(1, 0, 2).reshape(R, H)
    predictions_globals = _proj_log_softmax(main_flat, Wg, bg)
    predictions_senses = _proj_log_softmax(outs, Ws, bs)
    return (predictions_globals, predictions_senses)


# speedup vs baseline: 1.7585x; 1.0249x over previous
import functools

import jax
import jax.numpy as jnp
from jax import lax
from jax.experimental import pallas as pl
from jax.experimental.pallas import tpu as pltpu
from jax.experimental.pallas import tpu_sc as plsc

NUM_NODES = 60000
D = 300
B = 32
S = 35
H = 1150
R = B * S
NEG = -1e30


def _sc_gather(table, idx_padded, n_rows, n_cols):
    info = plsc.get_sparse_core_info()
    nw = info.num_cores * info.num_subcores
    per_w = n_rows // nw
    mesh = plsc.VectorSubcoreMesh(core_axis_name="c", subcore_axis_name="s")

    @functools.partial(
        pl.kernel,
        mesh=mesh,
        out_type=jax.ShapeDtypeStruct((n_rows, n_cols), jnp.float32),
        scratch_types=[
            pltpu.VMEM((per_w,), jnp.int32),
            pltpu.VMEM((per_w, n_cols), jnp.float32),
            pltpu.SemaphoreType.DMA,
        ],
    )
    def k(table_hbm, idx_hbm, out_hbm, idx_v, rows_v, sem):
        wid = lax.axis_index("s") * info.num_cores + lax.axis_index("c")
        base = wid * per_w
        pltpu.sync_copy(idx_hbm.at[pl.ds(base, per_w)], idx_v)
        pltpu.async_copy(table_hbm.at[idx_v], rows_v, sem).wait()
        pltpu.sync_copy(rows_v, out_hbm.at[pl.ds(base, per_w)])

    return k(table, idx_padded)


PAD_RT = 2000


def _pad_body(x_ref, out_ref):
    x = x_ref[...]
    out_ref[...] = jnp.concatenate(
        [x, jnp.zeros((PAD_RT, 384 - D), jnp.float32)], axis=1)


def _pad_table(X):
    n = X.shape[0]
    return pl.pallas_call(
        _pad_body,
        grid=(n // PAD_RT,),
        in_specs=[pl.BlockSpec((PAD_RT, D), lambda i: (i, 0))],
        out_specs=pl.BlockSpec((PAD_RT, 384), lambda i: (i, 0)),
        out_shape=jax.ShapeDtypeStruct((n, 384), jnp.float32),
    )(X)


def _dot(a, b):
    return lax.dot_general(a, b, (((1,), (1,)), ((), ())),
                           preferred_element_type=jnp.float32)


def _dotn(a, b):
    return lax.dot_general(a, b, (((1,), (0,)), ((), ())),
                           preferred_element_type=jnp.float32)


def _gru_body(x_ref, h0_ref, wi_ref, wh_ref, bi_ref, bh_ref,
              out_ref, g3_s, h_s):
    g3_s[...] = _dot(x_ref[...], wi_ref[...]) + bi_ref[...]
    h_s[...] = h0_ref[...]

    def step(t, carry):
        h = h_s[...]
        hb = h.astype(jnp.bfloat16)
        gh = _dotn(hb, wh_ref[...]) + bh_ref[...]
        row = pl.multiple_of(t * B, B)
        gi = g3_s[pl.ds(row, B), :]
        r = jax.nn.sigmoid(gi[:, 0:H] + gh[:, 0:H])
        z = jax.nn.sigmoid(gi[:, H:2 * H] + gh[:, H:2 * H])
        n = jnp.tanh(gi[:, 2 * H:3 * H] + r * gh[:, 2 * H:3 * H])
        hn = (1.0 - z) * n + z * h
        h_s[...] = hn
        out_ref[pl.ds(row, B), :] = hn.astype(jnp.bfloat16)
        return carry

    lax.fori_loop(0, S, step, 0)


def _gru_layer(x_bf, h0, w_ih, w_hh, b_ih, b_hh):
    wi = w_ih.astype(jnp.bfloat16)
    wh = w_hh.T.astype(jnp.bfloat16)
    bi = b_ih.reshape(1, 3 * H)
    bh = b_hh.reshape(1, 3 * H)
    return pl.pallas_call(
        _gru_body,
        out_shape=jax.ShapeDtypeStruct((R, H), jnp.bfloat16),
        scratch_shapes=[
            pltpu.VMEM((R, 3 * H), jnp.float32),
            pltpu.VMEM((B, H), jnp.float32),
        ],
    )(x_bf, h0, wi, wh, bi, bh)


def _gru_pair_body(x_ref, h0a_ref, h0b_ref, wia, wha, bia, bha,
                   wib, whb, bib, bhb, outa_ref, outb_ref,
                   g3a, g3b, ha_s, hb_s):
    x = x_ref[...]
    g3a[...] = (_dot(x, wia[...]) + bia[...]).astype(jnp.bfloat16)
    g3b[...] = (_dot(x, wib[...]) + bib[...]).astype(jnp.bfloat16)
    ha_s[...] = h0a_ref[...]
    hb_s[...] = h0b_ref[...]

    def step(t, carry):
        ha = ha_s[...]
        hb = hb_s[...]
        gha = _dotn(ha.astype(jnp.bfloat16), wha[...]) + bha[...]
        ghb = _dotn(hb.astype(jnp.bfloat16), whb[...]) + bhb[...]
        row = pl.multiple_of(t * B, B)
        gia = g3a[pl.ds(row, B), :].astype(jnp.float32)
        gib = g3b[pl.ds(row, B), :].astype(jnp.float32)
        ra = jax.nn.sigmoid(gia[:, 0:H] + gha[:, 0:H])
        za = jax.nn.sigmoid(gia[:, H:2 * H] + gha[:, H:2 * H])
        na = jnp.tanh(gia[:, 2 * H:3 * H] + ra * gha[:, 2 * H:3 * H])
        hna = (1.0 - za) * na + za * ha
        rb = jax.nn.sigmoid(gib[:, 0:H] + ghb[:, 0:H])
        zb = jax.nn.sigmoid(gib[:, H:2 * H] + ghb[:, H:2 * H])
        nb = jnp.tanh(gib[:, 2 * H:3 * H] + rb * ghb[:, 2 * H:3 * H])
        hnb = (1.0 - zb) * nb + zb * hb
        ha_s[...] = hna
        hb_s[...] = hnb
        outa_ref[pl.ds(row, B), :] = hna.astype(jnp.bfloat16)
        outb_ref[pl.ds(row, B), :] = hnb.astype(jnp.bfloat16)
        return carry

    lax.fori_loop(0, S, step, 0)


def _gru_pair(x_bf, h0a, h0b, wa_ih, wa_hh, ba_ih, ba_hh,
              wb_ih, wb_hh, bb_ih, bb_hh):
    outs = pl.pallas_call(
        _gru_pair_body,
        out_shape=[jax.ShapeDtypeStruct((R, H), jnp.bfloat16),
                   jax.ShapeDtypeStruct((R, H), jnp.bfloat16)],
        scratch_shapes=[
            pltpu.VMEM((R, 3 * H), jnp.bfloat16),
            pltpu.VMEM((R, 3 * H), jnp.bfloat16),
            pltpu.VMEM((B, H), jnp.float32),
            pltpu.VMEM((B, H), jnp.float32),
        ],
    )(x_bf, h0a, h0b,
      wa_ih.astype(jnp.bfloat16), wa_hh.T.astype(jnp.bfloat16),
      ba_ih.reshape(1, 3 * H), ba_hh.reshape(1, 3 * H),
      wb_ih.astype(jnp.bfloat16), wb_hh.T.astype(jnp.bfloat16),
      bb_ih.reshape(1, 3 * H), bb_hh.reshape(1, 3 * H))
    return outs


VT = 1024


def _proj_body(v_total, n_tiles, h_ref, w_ref, b_ref, logit_ref, lse_ref,
               m_s, s_s):
    i = pl.program_id(0)

    @pl.when(i == 0)
    def _():
        m_s[...] = jnp.full((R, 1), NEG, jnp.float32)
        s_s[...] = jnp.zeros((R, 1), jnp.float32)

    w = w_ref[...].astype(jnp.bfloat16)
    logits = _dot(h_ref[...], w) + b_ref[...]
    col = lax.broadcasted_iota(jnp.int32, (1, VT), 1) + i * VT
    masked = jnp.where(col < v_total, logits, NEG)
    tmax = jnp.max(masked, axis=1, keepdims=True)
    m_old = m_s[...]
    s_old = s_s[...]
    m_new = jnp.maximum(m_old, tmax)
    s_new = s_old * jnp.exp(m_old - m_new) + jnp.sum(
        jnp.exp(masked - m_new), axis=1, keepdims=True)
    m_s[...] = m_new
    s_s[...] = s_new
    logit_ref[...] = logits.astype(jnp.bfloat16)

    @pl.when(i == n_tiles - 1)
    def _():
        lse_ref[...] = m_new + jnp.log(s_new)


def _norm_body(logit_ref, lse_ref, out_ref):
    out_ref[...] = logit_ref[...].astype(jnp.float32) - lse_ref[...]


def _proj_log_softmax(h_bf, w, b):
    v_total = w.shape[0]
    n_tiles = pl.cdiv(v_total, VT)
    b2 = b.reshape(1, v_total)
    logits, lse = pl.pallas_call(
        functools.partial(_proj_body, v_total, n_tiles),
        grid=(n_tiles,),
        in_specs=[
            pl.BlockSpec((R, H), lambda i: (0, 0)),
            pl.BlockSpec((VT, H), lambda i: (i, 0)),
            pl.BlockSpec((1, VT), lambda i: (0, i)),
        ],
        out_specs=[
            pl.BlockSpec((R, VT), lambda i: (0, i)),
            pl.BlockSpec((R, 1), lambda i: (0, 0)),
        ],
        out_shape=[
            jax.ShapeDtypeStruct((R, v_total), jnp.bfloat16),
            jax.ShapeDtypeStruct((R, 1), jnp.float32),
        ],
        scratch_shapes=[
            pltpu.VMEM((R, 1), jnp.float32),
            pltpu.VMEM((R, 1), jnp.float32),
        ],
    )(h_bf, w, b2)
    return pl.pallas_call(
        _norm_body,
        grid=(n_tiles,),
        in_specs=[
            pl.BlockSpec((R, VT), lambda i: (0, i)),
            pl.BlockSpec((R, 1), lambda i: (0, 0)),
        ],
        out_specs=pl.BlockSpec((R, VT), lambda i: (0, i)),
        out_shape=jax.ShapeDtypeStruct((R, v_total), jnp.float32),
    )(logits, lse)


def kernel(batchinput_tensor, grapharea_matrix, X,
           W_ih_0, W_hh_0, b_ih_0, b_hh_0,
           W_ih_1, W_hh_1, b_ih_1, b_hh_1,
           W_ih_2, W_hh_2, b_ih_2, b_hh_2,
           W_ih_s, W_hh_s, b_ih_s, b_hh_s,
           Wg, bg, Ws, bs, memory_hn, memory_hn_senses):
    word_idx = batchinput_tensor[:, :, 0, 0].astype(jnp.int32)
    idx_sb = word_idx.T.reshape(-1)
    idx_pad = jnp.concatenate([idx_sb, jnp.zeros((1280 - R,), jnp.int32)])
    X_pad = _pad_table(X)
    emb = _sc_gather(X_pad, idx_pad, 1280, 384)[:R, :D]

    x = emb.astype(jnp.bfloat16)
    out0 = _gru_layer(x, memory_hn[0], W_ih_0, W_hh_0, b_ih_0, b_hh_0)
    out1, outs = _gru_pair(out0, memory_hn[1], memory_hn_senses[0],
                           W_ih_1, W_hh_1, b_ih_1, b_hh_1,
                           W_ih_s, W_hh_s, b_ih_s, b_hh_s)
    out2 = _gru_layer(out1, memory_hn[2], W_ih_2, W_hh_2, b_ih_2, b_hh_2)

    main_flat = out2.reshape(S, B, H).transpose(1, 0, 2).reshape(R, H)
    predictions_globals = _proj_log_softmax(main_flat, Wg, bg)
    predictions_senses = _proj_log_softmax(outs, Ws, bs)
    return (predictions_globals, predictions_senses)
